# baseline (speedup 1.0000x reference)
import functools

import jax
import jax.numpy as jnp
from jax import lax
from jax.experimental import pallas as pl
from jax.experimental.pallas import tpu as pltpu
from jax.experimental.pallas import tpu_sc as plsc

N = 50000
K = 64
NS = 16
NC = 2
CHUNK = 128
TILES = CHUNK // 16
ROWS_MAIN = 3128
ROWS_LAST = N - (NS - 1) * ROWS_MAIN
NCHUNKS = -(-ROWS_MAIN // CHUNK)


def _rsqrt_nr(q):
    i = plsc.bitcast(q, jnp.int32)
    i = jnp.int32(0x5F3759DF) - (i >> 1)
    y = plsc.bitcast(i, jnp.float32)
    for _ in range(3):
        y = y * (1.5 - 0.5 * q * y * y)
    return y


@functools.partial(
    pl.kernel,
    mesh=plsc.VectorSubcoreMesh(core_axis_name="c", subcore_axis_name="s"),
    out_type=jax.ShapeDtypeStruct((NC, NS, 2, 16), jnp.float32),
    compiler_params=pltpu.CompilerParams(needs_layout_passes=False),
    scratch_types=[
        pltpu.VMEM((CHUNK * K * 3,), jnp.float32),
        pltpu.VMEM((CHUNK * 3,), jnp.float32),
        pltpu.VMEM((CHUNK * 3,), jnp.float32),
        pltpu.VMEM((24 * 16,), jnp.float32),
        pltpu.VMEM((16,), jnp.float32),
        pltpu.VMEM((16,), jnp.float32),
    ],
)
def _sc_loss(params_hbm, sp_hbm, rp_hbm, rk_hbm, sk_hbm, rc_hbm, sc_hbm,
             out_hbm, patch_v, kpt_v, cor_v, par_v, sum_v, cnt_v):
    c = lax.axis_index("c")
    s = lax.axis_index("s")
    pltpu.sync_copy(params_hbm, par_v)
    iota = lax.iota(jnp.int32, 16)
    zero = jnp.zeros((16,), jnp.int32)

    def run_side(patch_hbm, kpt_hbm, cor_hbm, poff):
        P = [par_v[pl.ds((poff + j) * 16, 16)] for j in range(12)]
        base = s * ROWS_MAIN
        nrows = jnp.where(s == NS - 1, ROWS_LAST, ROWS_MAIN)
        end = base + nrows

        def chunk_body(ch, carry):
            acc_s, acc_c = carry
            row0 = base + ch * CHUNK
            row0c = jnp.minimum(row0, N - CHUNK)
            pltpu.sync_copy(patch_hbm.at[pl.ds(row0c * (K * 3), CHUNK * K * 3)],
                            patch_v)
            pltpu.sync_copy(kpt_hbm.at[pl.ds(row0c * 3, CHUNK * 3)], kpt_v)
            pltpu.sync_copy(cor_hbm.at[pl.ds(row0c * 3, CHUNK * 3)], cor_v)

            def tile_body(t, carry2):
                acc_s2, acc_c2 = carry2
                rloc = t * 16 + iota
                gid = row0c + rloc
                r3 = rloc * 3
                kx = plsc.load_gather(kpt_v, [r3])
                ky = plsc.load_gather(kpt_v, [r3 + 1])
                kz = plsc.load_gather(kpt_v, [r3 + 2])
                tx = P[0] * kx + P[1] * ky + P[2] * kz + P[3]
                ty = P[4] * kx + P[5] * ky + P[6] * kz + P[7]
                tz = P[8] * kx + P[9] * ky + P[10] * kz + P[11]
                rp = rloc * (K * 3)

                def k_body(kk, mv):
                    pbase = rp + kk * 3
                    px = plsc.load_gather(patch_v, [pbase])
                    py = plsc.load_gather(patch_v, [pbase + 1])
                    pz = plsc.load_gather(patch_v, [pbase + 2])
                    dx = px - tx
                    dy = py - ty
                    dz = pz - tz
                    return jnp.minimum(mv, dx * dx + dy * dy + dz * dz)

                minv = lax.fori_loop(0, K, k_body,
                                     jnp.full((16,), 1e30, jnp.float32))
                cx = plsc.load_gather(cor_v, [r3])
                cy = plsc.load_gather(cor_v, [r3 + 1])
                cz = plsc.load_gather(cor_v, [r3 + 2])
                dx = cx - tx
                dy = cy - ty
                dz = cz - tz
                q = dx * dx + dy * dy + dz * dz
                lossv = q * _rsqrt_nr(q)
                m = (gid >= row0) & (gid < end) & (minv < 1.0)
                return (acc_s2 + jnp.where(m, lossv, 0.0),
                        acc_c2 + jnp.where(m, 1.0, 0.0))

            return lax.fori_loop(0, TILES, tile_body, (acc_s, acc_c))

        acc_s, acc_c = lax.fori_loop(
            0, NCHUNKS, chunk_body,
            (jnp.zeros((16,), jnp.float32), jnp.zeros((16,), jnp.float32)))
        sum_v[...] = acc_s
        cnt_v[...] = acc_c
        pltpu.sync_copy(sum_v, out_hbm.at[c, s, 0])
        pltpu.sync_copy(cnt_v, out_hbm.at[c, s, 1])

    @pl.when(c == 0)
    def _():
        run_side(sp_hbm, rk_hbm, rc_hbm, 0)

    @pl.when(c == 1)
    def _():
        run_side(rp_hbm, sk_hbm, sc_hbm, 12)


def kernel(gt_transform, ref_kpts, src_kpts, src_patch_corr_kpts,
           ref_patch_corr_kpts, ref_corres, src_corres):
    inv_T = jnp.linalg.inv(gt_transform)

    def pvec(M):
        return jnp.concatenate(
            [M[0, :4], M[1, :4], M[2, :4]])

    params = jnp.concatenate([pvec(inv_T), pvec(gt_transform)])
    params = jnp.repeat(params[:, None], 16, axis=1).reshape(-1)
    out = _sc_loss(params,
                   src_patch_corr_kpts.reshape(-1),
                   ref_patch_corr_kpts.reshape(-1),
                   ref_kpts.reshape(-1), src_kpts.reshape(-1),
                   ref_corres.reshape(-1), src_corres.reshape(-1))
    sums = out[:, :, 0, :].sum(axis=(1, 2))
    cnts = out[:, :, 1, :].sum(axis=(1, 2))
    means = sums / cnts
    return (means[0] + means[1]) / 2.0
